# baseline (speedup 1.0000x reference)
import functools
import math

import jax
import jax.numpy as jnp
from jax import lax
from jax.experimental import pallas as pl
from jax.experimental.pallas import tpu as pltpu
from jax.experimental.pallas import tpu_sc as plsc

MODEL_DIM = 64
LANES = 16
NUM_CORES = 2
NUM_SUBCORES = 16
NW = NUM_CORES * NUM_SUBCORES
ROWS = 128
SCALE = 8.0


def _make_emb_kernel(B: int, D: int):
    assert B % (NW * ROWS) == 0
    n_ops = B // (NW * ROWS)

    mesh = plsc.VectorSubcoreMesh(core_axis_name="c", subcore_axis_name="s")

    @functools.partial(
        pl.kernel,
        mesh=mesh,
        out_type=jax.ShapeDtypeStruct((B, D), jnp.float32),
        compiler_params=pltpu.CompilerParams(use_tc_tiling_on_sc=False),
        scratch_types=[
            pltpu.VMEM((n_ops, ROWS), jnp.int32),
            pltpu.VMEM((ROWS, D), jnp.float32),
            pltpu.SemaphoreType.DMA,
        ],
    )
    def emb(table_hbm, idx_hbm, out_hbm, idx_v, rows_v, sem):
        wid = lax.axis_index("s") * NUM_CORES + lax.axis_index("c")
        base = wid * (n_ops * ROWS)
        pltpu.sync_copy(idx_hbm.at[wid], idx_v)

        def step(g, carry):
            pltpu.async_copy(table_hbm.at[idx_v.at[g]], rows_v, sem).wait()

            def scale_row(r, c2):
                for k in range(D // LANES):
                    sl = pl.ds(k * LANES, LANES)
                    rows_v[r, sl] = rows_v[r, sl] * SCALE
                return c2

            lax.fori_loop(0, ROWS, scale_row, 0)
            pltpu.sync_copy(rows_v, out_hbm.at[pl.ds(base + g * ROWS, ROWS)])
            return carry

        lax.fori_loop(0, n_ops, step, 0)

    return emb


def kernel(x, table):
    B = x.size
    D = table.shape[1]
    idx = x.astype(jnp.int32).reshape(NW, B // (NW * ROWS), ROWS)
    out = _make_emb_kernel(B, D)(table, idx)
    return out.reshape(x.shape + (D,))


# speedup vs baseline: 1.2054x; 1.2054x over previous
import functools
import math

import jax
import jax.numpy as jnp
from jax import lax
from jax.experimental import pallas as pl
from jax.experimental.pallas import tpu as pltpu
from jax.experimental.pallas import tpu_sc as plsc

MODEL_DIM = 64
LANES = 16
NUM_CORES = 2
NUM_SUBCORES = 16
NW = NUM_CORES * NUM_SUBCORES
ROWS = 128
NBUF = 8
HALF = 4
SCALE = 8.0


def _make_emb_kernel(B: int, D: int):
    assert B % (NW * ROWS) == 0
    n_ops = B // (NW * ROWS)
    assert n_ops % NBUF == 0 and n_ops // NBUF >= 2

    mesh = plsc.VectorSubcoreMesh(core_axis_name="c", subcore_axis_name="s")

    @functools.partial(
        pl.kernel,
        mesh=mesh,
        out_type=jax.ShapeDtypeStruct((B, D), jnp.float32),
        compiler_params=pltpu.CompilerParams(use_tc_tiling_on_sc=False),
        scratch_types=[
            pltpu.VMEM((n_ops, ROWS), jnp.int32),
            pltpu.VMEM((NBUF, ROWS, D), jnp.float32),
            pltpu.SemaphoreType.DMA((NBUF,)),
            pltpu.SemaphoreType.DMA((NBUF,)),
        ],
    )
    def emb(table_hbm, idx_hbm, out_hbm, idx_v, rows_v, gsem, wsem):
        wid = lax.axis_index("s") * NUM_CORES + lax.axis_index("c")
        base = wid * (n_ops * ROWS)
        pltpu.sync_copy(idx_hbm.at[wid], idx_v)

        def issue_gather(q, qb):
            pltpu.async_copy(table_hbm.at[idx_v.at[q]], rows_v.at[qb],
                             gsem.at[qb])

        def wait_gather(g, b):
            pltpu.make_async_copy(table_hbm.at[idx_v.at[g]], rows_v.at[b],
                                  gsem.at[b]).wait()

        def wait_writeback(b):
            pltpu.make_async_copy(rows_v.at[b], out_hbm.at[pl.ds(base, ROWS)],
                                  wsem.at[b]).wait()

        def scale_block(b):
            @plsc.parallel_loop(0, ROWS, unroll=4)
            def _(r):
                for k in range(D // LANES):
                    sl = pl.ds(k * LANES, LANES)
                    rows_v[b, r, sl] = rows_v[b, r, sl] * SCALE

        def process(g, b):
            wait_gather(g, b)
            scale_block(b)
            pltpu.async_copy(rows_v.at[b],
                             out_hbm.at[pl.ds(base + g * ROWS, ROWS)],
                             wsem.at[b])

        for q in range(HALF):
            issue_gather(q, q)

        for b in range(NBUF):
            q = b + HALF
            if q >= NBUF:
                wait_writeback(q % NBUF)
            issue_gather(q, q % NBUF)
            process(b, b)

        def outer(go, carry):
            g0 = go * NBUF
            for b in range(NBUF):
                qb = (b + HALF) % NBUF
                wait_writeback(qb)
                issue_gather(g0 + b + HALF, qb)
                process(g0 + b, b)
            return carry

        lax.fori_loop(1, n_ops // NBUF - 1, outer, 0)

        gl = n_ops - NBUF
        for b in range(HALF):
            qb = (b + HALF) % NBUF
            wait_writeback(qb)
            issue_gather(gl + b + HALF, qb)
            process(gl + b, b)
        for b in range(HALF, NBUF):
            process(gl + b, b)

        for b in range(NBUF):
            wait_writeback(b)

    return emb


def kernel(x, table):
    B = x.size
    D = table.shape[1]
    idx = x.astype(jnp.int32).reshape(NW, B // (NW * ROWS), ROWS)
    out = _make_emb_kernel(B, D)(table, idx)
    return out.reshape(x.shape + (D,))
